# baseline (speedup 1.0000x reference)
import functools

import jax
import jax.numpy as jnp
from jax.experimental import pallas as pl
from jax.experimental.pallas import tpu as pltpu


def _ae_kernel(x_ref,
               wih1, whh1, b1,
               wih2, whh2, b2,
               wih3, whh3, b3,
               wih4, whh4, b4,
               wout, bout,
               out_ref, seq_ref, xga_ref, xgb_ref, *, n_chunks, chunk, n_sub):
    T, BB, F = x_ref.shape
    HF = BB // n_sub

    def gates(g, c, H):
        i = 0.5 * jnp.tanh(g[:, :H]) + 0.5
        f = 0.5 * jnp.tanh(g[:, H:2 * H]) + 0.5
        gc = jnp.tanh(g[:, 2 * H:3 * H])
        o = 0.5 * jnp.tanh(g[:, 3 * H:]) + 0.5
        c = f * c + i * gc
        return c, o * jnp.tanh(c)

    def zstate(H):
        return tuple((jnp.zeros((HF, H), jnp.bfloat16),
                      jnp.zeros((HF, H), jnp.float32))
                     for _ in range(n_sub))

    def recurrent_step(row, read_g, whh, H, carry):
        gs = [read_g(s) + jnp.dot(carry[s][0], whh,
                                  preferred_element_type=jnp.float32)
              for s in range(n_sub)]
        new = []
        for s in range(n_sub):
            c, hf = gates(gs[s], carry[s][1], H)
            h = hf.astype(jnp.bfloat16)
            seq_ref[row, s * HF:(s + 1) * HF, :H] = h
            new.append((h, c))
        return tuple(new)

    def lstm_chunked(read_chunk, wih_ref, whh_ref, b_ref, finalize=None):
        wih = wih_ref[...]
        whh = whh_ref[...]
        b = b_ref[...]
        H = whh.shape[0]
        G = 4 * H
        def project(ci, buf):
            xg = jnp.dot(read_chunk(ci), wih,
                         preferred_element_type=jnp.float32) + b
            buf[:, :, :G] = xg.reshape(chunk, BB, G).astype(jnp.bfloat16)

        def steps(ci, buf, carry):
            for tl in range(chunk):
                def read_g(s, _tl=tl, _buf=buf):
                    return _buf[_tl, s * HF:(s + 1) * HF, :G].astype(
                        jnp.float32)

                carry = recurrent_step(ci * chunk + tl, read_g, whh, H, carry)

            if finalize is not None:
                wout, bout = finalize
                hs = seq_ref[pl.ds(ci * chunk, chunk), :, :H].reshape(
                    chunk * BB, H)
                y = jnp.dot(hs, wout[...],
                            preferred_element_type=jnp.float32) + bout[...]
                y = y.reshape(chunk, BB, F)
                for tl in range(chunk):
                    out_ref[:, ci * chunk + tl, :] = y[tl]
            return carry

        if n_chunks % 2 == 0:
            def pair_body(k, carry):
                c0 = 2 * k
                project(c0 + 1, xgb_ref)
                carry = steps(c0, xga_ref, carry)
                project(jnp.minimum(c0 + 2, n_chunks - 1), xga_ref)
                return steps(c0 + 1, xgb_ref, carry)

            project(0, xga_ref)
            carry = jax.lax.fori_loop(0, n_chunks // 2, pair_body, zstate(H))
        else:
            def chunk_body(ci, carry):
                project(ci, xga_ref)
                return steps(ci, xga_ref, carry)

            carry = jax.lax.fori_loop(0, n_chunks, chunk_body, zstate(H))
        return jnp.concatenate([hc[0] for hc in carry], axis=0)

    def lstm_repeated(xg_const, whh_ref):
        whh = whh_ref[...]
        H = whh.shape[0]
        xg_s = [xg_const[s * HF:(s + 1) * HF] for s in range(n_sub)]

        def step(uu, carry):
            for j in range(chunk):
                carry = recurrent_step(uu * chunk + j, lambda s: xg_s[s],
                                       whh, H, carry)
            return carry

        jax.lax.fori_loop(0, T // chunk, step, zstate(H))

    def read_x(ci):
        return x_ref[pl.ds(ci * chunk, chunk), :, :].reshape(chunk * BB, F)

    lstm_chunked(read_x, wih1, whh1, b1)
    H1 = whh1.shape[0]

    def read_h1(ci):
        return seq_ref[pl.ds(ci * chunk, chunk), :, :H1].reshape(
            chunk * BB, H1)

    h_last = lstm_chunked(read_h1, wih2, whh2, b2)

    xg3 = jnp.dot(h_last, wih3[...],
                  preferred_element_type=jnp.float32) + b3[...]
    lstm_repeated(xg3, whh3)
    H3 = whh3.shape[0]

    def read_h3(ci):
        return seq_ref[pl.ds(ci * chunk, chunk), :, :H3].reshape(
            chunk * BB, H3)

    lstm_chunked(read_h3, wih4, whh4, b4, finalize=(wout, bout))


def _combine_gates(whh_g):
    _, H, _ = whh_g.shape
    return jnp.transpose(whh_g, (1, 0, 2)).reshape(H, 4 * H)


@jax.jit
def kernel(data, p00, p01, p02, p03, p04, p05, p06, p07, p08, p09, p10,
           p11, p12, p13):
    B, T, F = data.shape
    BB = 256 if B % 256 == 0 else B
    n_sub = 2
    chunk = 16 if T % 16 == 0 else T
    n_chunks = T // chunk

    bf = jnp.bfloat16

    def halve_ifo(w):
        H = w.shape[-1] // 4
        scale = jnp.concatenate([jnp.full((1, H), s, w.dtype)
                                 for s in (0.5, 0.5, 1.0, 0.5)], axis=-1)
        return w * scale

    params = (halve_ifo(p00).astype(bf),
              halve_ifo(_combine_gates(p01)).astype(bf), halve_ifo(p02),
              halve_ifo(p03).astype(bf),
              halve_ifo(_combine_gates(p04)).astype(bf), halve_ifo(p05),
              halve_ifo(p06).astype(bf),
              halve_ifo(_combine_gates(p07)).astype(bf), halve_ifo(p08),
              halve_ifo(p09).astype(bf),
              halve_ifo(_combine_gates(p10)).astype(bf), halve_ifo(p11),
              p12.astype(bf), p13)

    h_max = max(p01.shape[2], p04.shape[2], p07.shape[2], p10.shape[2])
    x_tbf = jnp.transpose(data.astype(bf), (1, 0, 2))

    def whole(arr):
        return pl.BlockSpec(arr.shape, lambda b, _nd=arr.ndim: (0,) * _nd)

    return pl.pallas_call(
        functools.partial(_ae_kernel, n_chunks=n_chunks, chunk=chunk,
                          n_sub=n_sub),
        out_shape=jax.ShapeDtypeStruct((B, T, F), jnp.float32),
        grid=(B // BB,),
        in_specs=([pl.BlockSpec((T, BB, F), lambda b: (0, b, 0))]
                  + [whole(w) for w in params]),
        out_specs=pl.BlockSpec((BB, T, F), lambda b: (b, 0, 0)),
        scratch_shapes=[pltpu.VMEM((T, BB, h_max), jnp.bfloat16),
                        pltpu.VMEM((chunk, BB, 4 * h_max), jnp.bfloat16),
                        pltpu.VMEM((chunk, BB, 4 * h_max), jnp.bfloat16)],
        compiler_params=pltpu.CompilerParams(
            dimension_semantics=("parallel",),
            vmem_limit_bytes=64 * 1024 * 1024),
    )(x_tbf, *params)


# speedup vs baseline: 1.2083x; 1.2083x over previous
import functools

import jax
import jax.numpy as jnp
from jax.experimental import pallas as pl
from jax.experimental.pallas import tpu as pltpu


def _ae_kernel(x_ref,
               wih1, whh1, b1,
               wih2, whh2, b2,
               wih3, whh3, b3,
               wih4, whh4, b4,
               wout, bout,
               out_ref, seq_ref, xg_ref, *, n_chunks, chunk, n_sub):
    T, BB, F = x_ref.shape
    HF = BB // n_sub

    def gates(g, c, H):
        i = 0.5 * jnp.tanh(g[:, :H]) + 0.5
        f = 0.5 * jnp.tanh(g[:, H:2 * H]) + 0.5
        gc = jnp.tanh(g[:, 2 * H:3 * H])
        o = 0.5 * jnp.tanh(g[:, 3 * H:]) + 0.5
        c = f * c + i * gc
        return c, o * jnp.tanh(c)

    def zstate(H):
        return tuple((jnp.zeros((HF, H), jnp.bfloat16),
                      jnp.zeros((HF, H), jnp.float32))
                     for _ in range(n_sub))

    def recurrent_step(row, read_g, whh, H, carry):
        gs = [read_g(s) + jnp.dot(carry[s][0], whh,
                                  preferred_element_type=jnp.float32)
              for s in range(n_sub)]
        new = []
        for s in range(n_sub):
            c, hf = gates(gs[s], carry[s][1], H)
            h = hf.astype(jnp.bfloat16)
            seq_ref[row, s * HF:(s + 1) * HF, :H] = h
            new.append((h, c))
        return tuple(new)

    def lstm_chunked(read_chunk, wih_ref, whh_ref, b_ref, finalize=None):
        wih = wih_ref[...]
        whh = whh_ref[...]
        b = b_ref[...]
        H = whh.shape[0]
        G = 4 * H
        def chunk_body(ci, carry):
            xg = jnp.dot(read_chunk(ci), wih,
                         preferred_element_type=jnp.float32) + b
            xg_ref[:, :, :G] = xg.reshape(chunk, BB, G)

            for tl in range(chunk):
                def read_g(s, _tl=tl):
                    return xg_ref[_tl, s * HF:(s + 1) * HF, :G]

                carry = recurrent_step(ci * chunk + tl, read_g, whh, H, carry)

            if finalize is not None:
                wout, bout = finalize
                hs = seq_ref[pl.ds(ci * chunk, chunk), :, :H].reshape(
                    chunk * BB, H)
                y = jnp.dot(hs, wout[...],
                            preferred_element_type=jnp.float32) + bout[...]
                y = y.reshape(chunk, BB, F)
                for tl in range(chunk):
                    out_ref[:, ci * chunk + tl, :] = y[tl]
            return carry

        carry = jax.lax.fori_loop(0, n_chunks, chunk_body, zstate(H))
        return jnp.concatenate([hc[0] for hc in carry], axis=0)

    def lstm_repeated(xg_const, whh_ref):
        whh = whh_ref[...]
        H = whh.shape[0]
        xg_s = [xg_const[s * HF:(s + 1) * HF] for s in range(n_sub)]

        def step(uu, carry):
            for j in range(chunk):
                carry = recurrent_step(uu * chunk + j, lambda s: xg_s[s],
                                       whh, H, carry)
            return carry

        jax.lax.fori_loop(0, T // chunk, step, zstate(H))

    def read_x(ci):
        return x_ref[pl.ds(ci * chunk, chunk), :, :].reshape(chunk * BB, F)

    lstm_chunked(read_x, wih1, whh1, b1)
    H1 = whh1.shape[0]

    def read_h1(ci):
        return seq_ref[pl.ds(ci * chunk, chunk), :, :H1].reshape(
            chunk * BB, H1)

    h_last = lstm_chunked(read_h1, wih2, whh2, b2)

    xg3 = jnp.dot(h_last, wih3[...],
                  preferred_element_type=jnp.float32) + b3[...]
    lstm_repeated(xg3, whh3)
    H3 = whh3.shape[0]

    def read_h3(ci):
        return seq_ref[pl.ds(ci * chunk, chunk), :, :H3].reshape(
            chunk * BB, H3)

    lstm_chunked(read_h3, wih4, whh4, b4, finalize=(wout, bout))


def _combine_gates(whh_g):
    _, H, _ = whh_g.shape
    return jnp.transpose(whh_g, (1, 0, 2)).reshape(H, 4 * H)


@jax.jit
def kernel(data, p00, p01, p02, p03, p04, p05, p06, p07, p08, p09, p10,
           p11, p12, p13):
    B, T, F = data.shape
    BB = 256 if B % 256 == 0 else B
    n_sub = 2
    chunk = 16 if T % 16 == 0 else T
    n_chunks = T // chunk

    bf = jnp.bfloat16

    def halve_ifo(w):
        H = w.shape[-1] // 4
        scale = jnp.concatenate([jnp.full((1, H), s, w.dtype)
                                 for s in (0.5, 0.5, 1.0, 0.5)], axis=-1)
        return w * scale

    params = (halve_ifo(p00).astype(bf),
              halve_ifo(_combine_gates(p01)).astype(bf), halve_ifo(p02),
              halve_ifo(p03).astype(bf),
              halve_ifo(_combine_gates(p04)).astype(bf), halve_ifo(p05),
              halve_ifo(p06).astype(bf),
              halve_ifo(_combine_gates(p07)).astype(bf), halve_ifo(p08),
              halve_ifo(p09).astype(bf),
              halve_ifo(_combine_gates(p10)).astype(bf), halve_ifo(p11),
              p12.astype(bf), p13)

    h_max = max(p01.shape[2], p04.shape[2], p07.shape[2], p10.shape[2])
    x_tbf = jnp.transpose(data.astype(bf), (1, 0, 2))

    def whole(arr):
        return pl.BlockSpec(arr.shape, lambda b, _nd=arr.ndim: (0,) * _nd)

    return pl.pallas_call(
        functools.partial(_ae_kernel, n_chunks=n_chunks, chunk=chunk,
                          n_sub=n_sub),
        out_shape=jax.ShapeDtypeStruct((B, T, F), jnp.float32),
        grid=(B // BB,),
        in_specs=([pl.BlockSpec((T, BB, F), lambda b: (0, b, 0))]
                  + [whole(w) for w in params]),
        out_specs=pl.BlockSpec((BB, T, F), lambda b: (b, 0, 0)),
        scratch_shapes=[pltpu.VMEM((T, BB, h_max), jnp.bfloat16),
                        pltpu.VMEM((chunk, BB, 4 * h_max), jnp.float32)],
        compiler_params=pltpu.CompilerParams(
            dimension_semantics=("parallel",),
            vmem_limit_bytes=64 * 1024 * 1024),
    )(x_tbf, *params)
